# baseline (speedup 1.0000x reference)
import functools

import jax
import jax.numpy as jnp
from jax import lax
from jax.experimental import pallas as pl
from jax.experimental.pallas import tpu as pltpu
from jax.experimental.pallas import tpu_sc as plsc


def _make_lookup(n_tokens: int, d: int):
    info = plsc.get_sparse_core_info()
    nw = info.num_cores * info.num_subcores
    assert n_tokens % (8 * nw) == 0 and d % 32 == 0
    n_per_w = n_tokens // nw
    chunk = 32
    while n_per_w % (2 * chunk):
        chunk //= 2
    n_chunks = n_per_w // chunk
    mesh = plsc.VectorSubcoreMesh(core_axis_name="c", subcore_axis_name="s")

    @functools.partial(
        pl.kernel,
        mesh=mesh,
        out_type=jax.ShapeDtypeStruct((n_tokens, d), jnp.uint32),
        scratch_types=[
            pltpu.VMEM((n_per_w,), jnp.int32),
            pltpu.VMEM((chunk, d // 2), jnp.uint32),
            pltpu.VMEM((chunk, d // 2), jnp.uint32),
            pltpu.VMEM((chunk, d), jnp.uint32),
            pltpu.VMEM((chunk, d), jnp.uint32),
            pltpu.SemaphoreType.DMA,
            pltpu.SemaphoreType.DMA,
        ],
    )
    def lookup(tab_hbm, idx_hbm, out_hbm, idx_v, g0, g1, f0, f1, gsem, wsem):
        wid = lax.axis_index("s") * info.num_cores + lax.axis_index("c")
        base = wid * n_per_w
        pltpu.sync_copy(idx_hbm.at[pl.ds(base, n_per_w)], idx_v)

        gbufs = (g0, g1)
        fbufs = (f0, f1)

        def start_gather(ci, b):
            pltpu.async_copy(
                tab_hbm.at[idx_v.at[pl.ds(ci * chunk, chunk)]], gbufs[b], gsem
            )

        def drain(ref, sem):
            pltpu.make_async_copy(out_hbm.at[pl.ds(0, chunk)], ref, sem).wait()

        def convert(gb, fb):
            @plsc.parallel_loop(0, chunk, unroll=8)
            def crow(r):
                for j in range(d // 32):
                    w = gb[r, pl.ds(j * 16, 16)]
                    fb[r, pl.ds(j * 32, 16)] = w << 16
                    fb[r, pl.ds(j * 32 + 16, 16)] = (w >> 16) << 16

        start_gather(0, 0)
        start_gather(1, 1)

        def body(g, _):
            for b in range(2):
                ci = g * 2 + b
                pltpu.make_async_copy(
                    tab_hbm.at[pl.ds(0, chunk)], gbufs[b], gsem
                ).wait()

                @pl.when(ci >= 2)
                def _():
                    drain(fbufs[b], wsem)

                convert(gbufs[b], fbufs[b])

                @pl.when(ci + 2 < n_chunks)
                def _():
                    start_gather(ci + 2, b)

                pltpu.async_copy(
                    fbufs[b], out_hbm.at[pl.ds(base + ci * chunk, chunk)], wsem
                )
            return 0

        lax.fori_loop(0, n_chunks // 2, body, 0)
        drain(fbufs[0], wsem)
        drain(fbufs[1], wsem)

    return lookup


def kernel(tokens, attention_mask, table):
    b, s = tokens.shape
    vocab, d = table.shape
    idx = tokens.reshape(b * s).astype(jnp.int32)
    tabp = (
        table.astype(jnp.bfloat16)
        .reshape(vocab, d // 32, 2, 16)
        .transpose(0, 1, 3, 2)
        .reshape(vocab, d // 2, 2)
    )
    tab_u32 = jax.lax.bitcast_convert_type(tabp, jnp.uint32)
    out = _make_lookup(b * s, d)(tab_u32, idx)
    return jax.lax.bitcast_convert_type(out, jnp.float32).reshape(b, s, d)


# speedup vs baseline: 1.5538x; 1.5538x over previous
import functools

import jax
import jax.numpy as jnp
from jax import lax
from jax.experimental import pallas as pl
from jax.experimental.pallas import tpu as pltpu
from jax.experimental.pallas import tpu_sc as plsc


def _make_lookup(n_tokens: int, d: int):
    info = plsc.get_sparse_core_info()
    nw = info.num_cores * info.num_subcores
    assert n_tokens % (8 * nw) == 0
    n_per_w = n_tokens // nw
    chunk = 32
    while n_per_w % (4 * chunk):
        chunk //= 2
    n_chunks = n_per_w // chunk
    mesh = plsc.VectorSubcoreMesh(core_axis_name="c", subcore_axis_name="s")

    @functools.partial(
        pl.kernel,
        mesh=mesh,
        out_type=jax.ShapeDtypeStruct((n_tokens, d), jnp.float32),
        scratch_types=[
            pltpu.VMEM((n_per_w,), jnp.int32),
            pltpu.VMEM((chunk, d), jnp.float32),
            pltpu.VMEM((chunk, d), jnp.float32),
            pltpu.VMEM((chunk, d), jnp.float32),
            pltpu.VMEM((chunk, d), jnp.float32),
            pltpu.SemaphoreType.DMA,
            pltpu.SemaphoreType.DMA,
        ],
    )
    def lookup(table_hbm, idx_hbm, out_hbm, idx_v, b0, b1, b2, b3, gsem, wsem):
        wid = lax.axis_index("s") * info.num_cores + lax.axis_index("c")
        base = wid * n_per_w
        pltpu.sync_copy(idx_hbm.at[pl.ds(base, n_per_w)], idx_v)

        bufs = (b0, b1, b2, b3)

        def start_gather(ci, b):
            pltpu.async_copy(
                table_hbm.at[idx_v.at[pl.ds(ci * chunk, chunk)]], bufs[b], gsem
            )

        def drain(ref, sem):
            pltpu.make_async_copy(table_hbm.at[pl.ds(0, chunk)], ref, sem).wait()

        start_gather(0, 0)
        start_gather(1, 1)

        def body(g, _):
            for b in range(4):
                ci = g * 4 + b
                drain(bufs[b], gsem)
                pltpu.async_copy(
                    bufs[b], out_hbm.at[pl.ds(base + ci * chunk, chunk)], wsem
                )

                @pl.when(ci >= 2)
                def _():
                    drain(bufs[(b + 2) % 4], wsem)

                @pl.when(ci + 2 < n_chunks)
                def _():
                    start_gather(ci + 2, (b + 2) % 4)
            return 0

        lax.fori_loop(0, n_chunks // 4, body, 0)
        drain(bufs[(n_chunks - 2) % 4], wsem)
        drain(bufs[(n_chunks - 1) % 4], wsem)

    return lookup


def kernel(tokens, attention_mask, table):
    b, s = tokens.shape
    d = table.shape[1]
    idx = tokens.reshape(b * s).astype(jnp.int32)
    out = _make_lookup(b * s, d)(table, idx)
    return out.reshape(b, s, d)
